# Initial kernel scaffold; baseline (speedup 1.0000x reference)
import jax
import jax.numpy as jnp
from jax.experimental import pallas as pl


def kernel(x, edge_index, idx, W1, b1, W2, b2, Wd, bd):
    raise NotImplementedError("write your pallas kernel here")


# speedup vs baseline: 4.6788x; 4.6788x over previous
import functools

import jax
import jax.numpy as jnp
from jax import lax
from jax.experimental import pallas as pl
from jax.experimental.pallas import tpu as pltpu
from jax.experimental.pallas import tpu_sc as plsc

N = 10000
E = 160000
D = 256
H = 256
C = 40
CP = 64
NIDX = 5000
NIDXP = 5120

NC = 2
NS = 16
ROWS_PER_TILE = N // NS
EDGES_PER_TILE = E // NS
ECHUNK = 80
NCHUNKS = EDGES_PER_TILE // ECHUNK
EDGES_PER_TILE2 = E // (NC * NS)
DCHUNK = 40
DCHUNKS = EDGES_PER_TILE2 // DCHUNK

def _mesh():
    return plsc.VectorSubcoreMesh(core_axis_name="c", subcore_axis_name="s",
                                  num_cores=NC, num_subcores=NS)


_ROWA = 632
_ROWB = N - (NS - 1) * _ROWA


def _tile_rows(s, emit):
    @pl.when(s < NS - 1)
    def _():
        emit(s * _ROWA, _ROWA)

    @pl.when(s == NS - 1)
    def _():
        emit((NS - 1) * _ROWA, _ROWB)


def _deg_body(dst_hbm, zeros16_hbm, hist_hbm, ones_v, didx, acc, sem):
    c = lax.axis_index("c")
    s = lax.axis_index("s")
    w = c * NS + s
    one = jnp.ones((16,), jnp.float32)
    for r in range(DCHUNK):
        ones_v[r, :] = one
    _tile_rows(s, lambda off, n: pltpu.sync_copy(
        zeros16_hbm.at[pl.ds(off, n)], acc.at[pl.ds(off, n)]))
    plsc.subcore_barrier()

    def body(k, _):
        eb = w * EDGES_PER_TILE2 + k * DCHUNK
        pltpu.sync_copy(dst_hbm.at[pl.ds(eb, DCHUNK)], didx)
        pltpu.sync_copy(ones_v, acc.at[didx], add=True)
        return _

    lax.fori_loop(0, DCHUNKS, body, None)
    plsc.subcore_barrier()
    _tile_rows(s, lambda off, n: pltpu.sync_copy(
        acc.at[pl.ds(off, n)], hist_hbm.at[pl.ds(c * N + off, n)]))


def _make_deg_kernel():
    return pl.kernel(
        _deg_body,
        out_type=jax.ShapeDtypeStruct((2 * N, 16), jnp.float32),
        mesh=_mesh(),
        scratch_types=[
            pltpu.VMEM((DCHUNK, 16), jnp.float32),
            pltpu.VMEM((DCHUNK,), jnp.int32),
            pltpu.VMEM_SHARED((N, 16), jnp.float32),
            pltpu.SemaphoreType.DMA,
        ],
    )


def _scat_body(hw, g_hbm, src_hbm, dst_hbm, o_hbm, sidx, didx, rows, acc, sem):
    c = lax.axis_index("c")
    s = lax.axis_index("s")
    roff = c * N
    _tile_rows(s, lambda off, n: pltpu.sync_copy(
        g_hbm.at[pl.ds(roff + off, n)], acc.at[pl.ds(off, n)]))
    plsc.subcore_barrier()

    def body(k, _):
        eb = s * EDGES_PER_TILE + k * ECHUNK
        pltpu.sync_copy(src_hbm.at[pl.ds(eb, ECHUNK)], sidx)
        pltpu.sync_copy(dst_hbm.at[pl.ds(eb, ECHUNK)], didx)
        for j in range(ECHUNK // 16):
            sidx[pl.ds(j * 16, 16)] = sidx[pl.ds(j * 16, 16)] + roff
        pltpu.async_copy(g_hbm.at[sidx], rows, sem).wait()
        pltpu.sync_copy(rows, acc.at[didx], add=True)
        return _

    lax.fori_loop(0, NCHUNKS, body, None)
    plsc.subcore_barrier()
    _tile_rows(s, lambda off, n: pltpu.sync_copy(
        acc.at[pl.ds(off, n)], o_hbm.at[pl.ds(roff + off, n)]))


def _make_scat_kernel(hw):
    return pl.kernel(
        functools.partial(_scat_body, hw),
        out_type=jax.ShapeDtypeStruct((2 * N, hw), jnp.float32),
        mesh=_mesh(),
        scratch_types=[
            pltpu.VMEM((ECHUNK,), jnp.int32),
            pltpu.VMEM((ECHUNK,), jnp.int32),
            pltpu.VMEM((ECHUNK, hw), jnp.float32),
            pltpu.VMEM_SHARED((N, hw), jnp.float32),
            pltpu.SemaphoreType.DMA,
        ],
    )


def _scat3_body(g_hbm, src_hbm, dst_hbm, z128_hbm, o_hbm,
                sidx, didx, rows, acc, sem):
    c = lax.axis_index("c")
    s = lax.axis_index("s")

    @pl.when(c == 0)
    def _():
        _tile_rows(s, lambda off, n: pltpu.sync_copy(
            g_hbm.at[pl.ds(off, n)], acc.at[pl.ds(off, n)]))

    @pl.when(c == 1)
    def _():
        _tile_rows(s, lambda off, n: pltpu.sync_copy(
            z128_hbm.at[pl.ds(off, n)], acc.at[pl.ds(off, n)]))

    plsc.subcore_barrier()
    w = c * NS + s

    def body(k, _):
        eb = w * EDGES_PER_TILE2 + k * DCHUNK
        pltpu.sync_copy(src_hbm.at[pl.ds(eb, DCHUNK)], sidx)
        pltpu.sync_copy(dst_hbm.at[pl.ds(eb, DCHUNK)], didx)
        pltpu.async_copy(g_hbm.at[sidx], rows, sem).wait()
        pltpu.sync_copy(rows, acc.at[didx], add=True)
        return _

    lax.fori_loop(0, DCHUNKS, body, None)
    plsc.subcore_barrier()
    _tile_rows(s, lambda off, n: pltpu.sync_copy(
        acc.at[pl.ds(off, n)], o_hbm.at[pl.ds(c * N + off, n)]))


def _make_scat3_kernel():
    return pl.kernel(
        _scat3_body,
        out_type=jax.ShapeDtypeStruct((2 * N, 128), jnp.float32),
        mesh=_mesh(),
        scratch_types=[
            pltpu.VMEM((DCHUNK,), jnp.int32),
            pltpu.VMEM((DCHUNK,), jnp.int32),
            pltpu.VMEM((DCHUNK, 128), jnp.float32),
            pltpu.VMEM_SHARED((N, 128), jnp.float32),
            pltpu.SemaphoreType.DMA,
        ],
    )


def _gather_body(lsm_hbm, idx_hbm, out_hbm, iv, rows, sem):
    c = lax.axis_index("c")
    s = lax.axis_index("s")
    w = c * NS + s
    for j in range(2):
        b = w * 160 + j * 80
        pltpu.sync_copy(idx_hbm.at[pl.ds(b, 80)], iv)
        pltpu.async_copy(lsm_hbm.at[iv], rows, sem).wait()
        pltpu.sync_copy(rows, out_hbm.at[pl.ds(b, 80)])


def _make_gather_kernel():
    return pl.kernel(
        _gather_body,
        out_type=jax.ShapeDtypeStruct((NIDXP, 128), jnp.float32),
        mesh=_mesh(),
        scratch_types=[
            pltpu.VMEM((80,), jnp.int32),
            pltpu.VMEM((80, 128), jnp.float32),
            pltpu.SemaphoreType.DMA,
        ],
    )


_R = 1000
_NR = N // _R


def _ds_of(hista, histb):
    deg = hista[:, 0:1] + histb[:, 0:1] + 1.0
    return lax.rsqrt(deg)


def _mm1_body(x_ref, w_ref, b_ref, ha_ref, hb_ref, out_ref):
    ds = _ds_of(ha_ref[...], hb_ref[...])
    h = jnp.dot(x_ref[...], w_ref[0],
                preferred_element_type=jnp.float32) + b_ref[0]
    out_ref[...] = ds * h


def _mm2_body(oa_ref, ob_ref, w_ref, b_ref, ha_ref, hb_ref, out_ref):
    ds = _ds_of(ha_ref[...], hb_ref[...])
    t = jax.nn.relu(ds * jnp.concatenate([oa_ref[...], ob_ref[...]], axis=1))
    h = jnp.dot(t, w_ref[0], preferred_element_type=jnp.float32) + b_ref[0]
    out_ref[...] = ds * h


def _mm3_body(oa_ref, ob_ref, w_ref, b_ref, ha_ref, hb_ref, out_ref):
    ds = _ds_of(ha_ref[...], hb_ref[...])
    t = jax.nn.relu(ds * jnp.concatenate([oa_ref[...], ob_ref[...]], axis=1))
    h = jnp.dot(t, w_ref[...], preferred_element_type=jnp.float32) + b_ref[...]
    out_ref[...] = jnp.concatenate(
        [ds * h, jnp.zeros((h.shape[0], 128 - CP), jnp.float32)], axis=1)


def _finish_body(oa_ref, ob_ref, ha_ref, hb_ref, out_ref):
    ds = _ds_of(ha_ref[...], hb_ref[...])
    zc = ds * (oa_ref[...][:, :C] + ob_ref[...][:, :C])
    m = jnp.max(zc, axis=1, keepdims=True)
    lse = jnp.log(jnp.sum(jnp.exp(zc - m), axis=1, keepdims=True)) + m
    out_ref[...] = jnp.concatenate(
        [zc - lse, jnp.zeros((zc.shape[0], 128 - C), jnp.float32)], axis=1)


def _hist_specs():
    return [
        pl.BlockSpec((_R, 16), lambda i, c: (i, 0)),
        pl.BlockSpec((_R, 16), lambda i, c: (_NR + i, 0)),
    ]


def _split_wb(w, b, hcols):
    hh = hcols // 2
    d = w.shape[0]
    wh = jnp.stack([w[:, :hh], w[:, hh:]], axis=0)
    bh = b.reshape(1, hcols)
    bh = jnp.stack([bh[:, :hh], bh[:, hh:]], axis=0)
    return wh, bh, hh, d


def _mm1_call(x, w, b, hist):
    wh, bh, hh, d = _split_wb(w, b, H)
    return pl.pallas_call(
        _mm1_body,
        grid=(_NR, 2),
        in_specs=[
            pl.BlockSpec((_R, D), lambda i, c: (i, 0)),
            pl.BlockSpec((1, d, hh), lambda i, c: (c, 0, 0)),
            pl.BlockSpec((1, 1, hh), lambda i, c: (c, 0, 0)),
            *_hist_specs(),
        ],
        out_specs=pl.BlockSpec((_R, hh), lambda i, c: (c * _NR + i, 0)),
        out_shape=jax.ShapeDtypeStruct((2 * N, hh), jnp.float32),
    )(x, wh, bh, hist, hist)


def _mm2_call(o, w, b, hist, hcols):
    wh, bh, hh, d = _split_wb(w, b, hcols)
    return pl.pallas_call(
        _mm2_body,
        grid=(_NR, 2),
        in_specs=[
            pl.BlockSpec((_R, H // 2), lambda i, c: (i, 0)),
            pl.BlockSpec((_R, H // 2), lambda i, c: (_NR + i, 0)),
            pl.BlockSpec((1, d, hh), lambda i, c: (c, 0, 0)),
            pl.BlockSpec((1, 1, hh), lambda i, c: (c, 0, 0)),
            *_hist_specs(),
        ],
        out_specs=pl.BlockSpec((_R, hh), lambda i, c: (c * _NR + i, 0)),
        out_shape=jax.ShapeDtypeStruct((2 * N, hh), jnp.float32),
    )(o, o, wh, bh, hist, hist)


def _mm3_call(o, w, b, hist):
    return pl.pallas_call(
        _mm3_body,
        grid=(_NR,),
        in_specs=[
            pl.BlockSpec((_R, H // 2), lambda i: (i, 0)),
            pl.BlockSpec((_R, H // 2), lambda i: (_NR + i, 0)),
            pl.BlockSpec((D, CP), lambda i: (0, 0)),
            pl.BlockSpec((1, CP), lambda i: (0, 0)),
            pl.BlockSpec((_R, 16), lambda i: (i, 0)),
            pl.BlockSpec((_R, 16), lambda i: (_NR + i, 0)),
        ],
        out_specs=pl.BlockSpec((_R, 128), lambda i: (i, 0)),
        out_shape=jax.ShapeDtypeStruct((N, 128), jnp.float32),
    )(o, o, w, b, hist, hist)


def _finish_call(o, hist):
    return pl.pallas_call(
        _finish_body,
        grid=(_NR,),
        in_specs=[
            pl.BlockSpec((_R, 128), lambda i: (i, 0)),
            pl.BlockSpec((_R, 128), lambda i: (_NR + i, 0)),
            pl.BlockSpec((_R, 16), lambda i: (i, 0)),
            pl.BlockSpec((_R, 16), lambda i: (_NR + i, 0)),
        ],
        out_specs=pl.BlockSpec((_R, 128), lambda i: (i, 0)),
        out_shape=jax.ShapeDtypeStruct((N, 128), jnp.float32),
    )(o, o, hist, hist)


def kernel(x, edge_index, idx, W1, b1, W2, b2, Wd, bd):
    src = edge_index[0]
    dst = edge_index[1]
    zeros16 = jnp.zeros((N, 16), jnp.float32)
    zeros128 = jnp.zeros((N, 128), jnp.float32)
    Wdp = jnp.pad(Wd, ((0, 0), (0, CP - C)))
    bdp = jnp.pad(bd, (0, CP - C)).reshape(1, CP)
    idxp = jnp.pad(idx, (0, NIDXP - NIDX))

    hist = _make_deg_kernel()(dst, zeros16)

    scat_h = _make_scat_kernel(H // 2)

    g1 = _mm1_call(x, W1, b1, hist)
    o1 = scat_h(g1, src, dst)
    g2 = _mm2_call(o1, W2, b2, hist, H)
    o2 = scat_h(g2, src, dst)
    g3 = _mm3_call(o2, Wdp, bdp, hist)
    o3 = _make_scat3_kernel()(g3, src, dst, zeros128)
    lsm = _finish_call(o3, hist)
    out = _make_gather_kernel()(lsm, idxp)
    return out[:NIDX, :C]


# speedup vs baseline: 11.5591x; 2.4705x over previous
import jax
import jax.numpy as jnp
from jax import lax
from jax.experimental import pallas as pl
from jax.experimental.pallas import tpu as pltpu
from jax.experimental.pallas import tpu_sc as plsc

N = 10000
E = 160000
D = 256
H = 256
HH = H // 2
C = 40
CP = 64
NIDX = 5000
NIDXP = 5120

NC = 2
NS = 16
CH = 80
CHH = 128

EPT1 = E // NS
FULL1, REM1 = EPT1 // CH, EPT1 % CH
EPT2 = E // (NC * NS)
FULL2, REM2 = EPT2 // CH, EPT2 % CH
FULLH, REMH = EPT2 // CHH, EPT2 % CHH


def _mesh():
    return plsc.VectorSubcoreMesh(core_axis_name="c", subcore_axis_name="s",
                                  num_cores=NC, num_subcores=NS)


_ROWA = 632
_ROWB = N - (NS - 1) * _ROWA


def _tile_rows(s, emit):
    @pl.when(s < NS - 1)
    def _():
        emit(s * _ROWA, _ROWA)

    @pl.when(s == NS - 1)
    def _():
        emit((NS - 1) * _ROWA, _ROWB)


def _deg_body(dst_hbm, zeros16_hbm, hist_hbm, *scratch):
    c = lax.axis_index("c")
    s = lax.axis_index("s")
    w = c * NS + s
    e0 = w * EPT2
    ones_v = scratch[0]
    didx = scratch[1:1 + _NB]
    didxr = scratch[1 + _NB]
    semi = scratch[2 + _NB:2 + 2 * _NB]
    sems = scratch[2 + 2 * _NB:2 + 3 * _NB]
    acc = scratch[-1]
    one = jnp.ones((16,), jnp.float32)
    for r in range(CHH):
        ones_v[r, :] = one
    _tile_rows(s, lambda off, n: pltpu.sync_copy(
        zeros16_hbm.at[pl.ds(off, n)], acc.at[pl.ds(off, n)]))
    plsc.subcore_barrier()

    def issue_idx(k, b):
        pltpu.async_copy(dst_hbm.at[pl.ds(e0 + k * CHH, CHH)], didx[b], semi[b])

    def wait_idx(b):
        pltpu.make_async_copy(dst_hbm.at[pl.ds(0, CHH)], didx[b], semi[b]).wait()

    def wait_scat(b):
        pltpu.make_async_copy(ones_v, acc.at[didx[b]], sems[b]).wait()

    def step(k, b):
        wait_idx(b)
        pltpu.async_copy(ones_v, acc.at[didx[b]], sems[b], add=True)
        b2 = (b + 2) % _NB

        @pl.when(jnp.logical_and(k + 2 < FULLH, k >= 2))
        def _():
            wait_scat(b2)

        @pl.when(k + 2 < FULLH)
        def _():
            issue_idx(k + 2, b2)

    issue_idx(0, 0)
    issue_idx(1, 1)
    grp, tail = FULLH // _NB, FULLH % _NB

    def body(j, carry):
        for b in range(_NB):
            step(j * _NB + b, b)
        return carry

    lax.fori_loop(0, grp, body, None)
    for t in range(tail):
        k = grp * _NB + t
        step(k, k % _NB)

    if REMH:
        pltpu.sync_copy(dst_hbm.at[pl.ds(e0 + FULLH * CHH, REMH)], didxr)
        pltpu.sync_copy(ones_v.at[pl.ds(0, REMH)], acc.at[didxr], add=True)
    wait_scat((FULLH - 2) % _NB)
    wait_scat((FULLH - 1) % _NB)

    plsc.subcore_barrier()
    _tile_rows(s, lambda off, n: pltpu.sync_copy(
        acc.at[pl.ds(off, n)], hist_hbm.at[pl.ds(c * N + off, n)]))


def _make_deg_kernel():
    return pl.kernel(
        _deg_body,
        out_type=jax.ShapeDtypeStruct((2 * N, 16), jnp.float32),
        mesh=_mesh(),
        scratch_types=(
            [pltpu.VMEM((CHH, 16), jnp.float32)]
            + [pltpu.VMEM((CHH,), jnp.int32) for _ in range(_NB)]
            + [pltpu.VMEM((REMH,), jnp.int32)]
            + [pltpu.SemaphoreType.DMA for _ in range(2 * _NB)]
            + [pltpu.VMEM_SHARED((N, 16), jnp.float32)]
        ),
    )


_NB = 4


def _edge_pipe(tbl, acc, src_hbm, dst_hbm, e0, nfull, rem, scr):
    sidx = scr[0:_NB]
    didx = scr[_NB:2 * _NB]
    rows = scr[2 * _NB:3 * _NB]
    sidxr, didxr, rowsr = scr[3 * _NB:3 * _NB + 3]
    semi = scr[3 * _NB + 3:3 * _NB + 3 + _NB]
    semg = scr[3 * _NB + 3 + _NB:3 * _NB + 3 + 2 * _NB]
    sems = scr[3 * _NB + 3 + 2 * _NB:3 * _NB + 3 + 3 * _NB]

    def issue_idx(k, b):
        pltpu.async_copy(src_hbm.at[pl.ds(e0 + k * CH, CH)], sidx[b], semi[b])
        pltpu.async_copy(dst_hbm.at[pl.ds(e0 + k * CH, CH)], didx[b], semi[b])

    def wait_idx(b):
        pltpu.make_async_copy(src_hbm.at[pl.ds(0, CH)], sidx[b], semi[b]).wait()
        pltpu.make_async_copy(dst_hbm.at[pl.ds(0, CH)], didx[b], semi[b]).wait()

    def issue_gather(b):
        pltpu.async_copy(tbl.at[sidx[b]], rows[b], semg[b])

    def wait_gather(b):
        pltpu.make_async_copy(tbl.at[pl.ds(0, CH)], rows[b], semg[b]).wait()

    def issue_scat(b):
        pltpu.async_copy(rows[b], acc.at[didx[b]], sems[b], add=True)

    def wait_scat(b):
        pltpu.make_async_copy(rows[b], acc.at[didx[b]], sems[b]).wait()

    def step(k, b):
        wait_gather(b)
        issue_scat(b)
        b2 = (b + 2) % _NB
        b1 = (b + 1) % _NB

        @pl.when(jnp.logical_and(k + 2 < nfull, k >= 2))
        def _():
            wait_scat(b2)

        @pl.when(k + 2 < nfull)
        def _():
            issue_idx(k + 2, b2)

        @pl.when(k + 1 < nfull)
        def _():
            wait_idx(b1)
            issue_gather(b1)

    issue_idx(0, 0)
    issue_idx(1, 1)
    wait_idx(0)
    issue_gather(0)

    grp, tail = nfull // _NB, nfull % _NB

    def body(j, carry):
        for b in range(_NB):
            step(j * _NB + b, b)
        return carry

    lax.fori_loop(0, grp, body, None)
    for t in range(tail):
        k = grp * _NB + t
        step(k, k % _NB)

    if rem:
        eb = e0 + nfull * CH
        pltpu.sync_copy(src_hbm.at[pl.ds(eb, rem)], sidxr)
        pltpu.sync_copy(dst_hbm.at[pl.ds(eb, rem)], didxr)
        pltpu.async_copy(tbl.at[sidxr], rowsr, semg[0]).wait()
        pltpu.sync_copy(rowsr, acc.at[didxr], add=True)

    wait_scat((nfull - 2) % _NB)
    wait_scat((nfull - 1) % _NB)


def _scat_scratch(rem):
    r = max(rem, 8)
    return (
        [pltpu.VMEM((CH,), jnp.int32) for _ in range(2 * _NB)]
        + [pltpu.VMEM((CH, HH), jnp.float32) for _ in range(_NB)]
        + [pltpu.VMEM((r,), jnp.int32),
           pltpu.VMEM((r,), jnp.int32),
           pltpu.VMEM((r, HH), jnp.float32)]
        + [pltpu.SemaphoreType.DMA for _ in range(3 * _NB)]
        + [pltpu.VMEM_SHARED((N, HH), jnp.float32)]
    )


def _scat_one(tbl, out, src_hbm, dst_hbm, s, acc, scr):
    _tile_rows(s, lambda off, n: pltpu.sync_copy(
        tbl.at[pl.ds(off, n)], acc.at[pl.ds(off, n)]))
    plsc.subcore_barrier()
    _edge_pipe(tbl, acc, src_hbm, dst_hbm, s * EPT1, FULL1, REM1, scr)
    plsc.subcore_barrier()
    _tile_rows(s, lambda off, n: pltpu.sync_copy(
        acc.at[pl.ds(off, n)], out.at[pl.ds(off, n)]))


def _scat_body(ga, gb, src_hbm, dst_hbm, oa, ob, *scratch):
    c = lax.axis_index("c")
    s = lax.axis_index("s")
    acc = scratch[-1]
    scr = scratch[:-1]

    @pl.when(c == 0)
    def _():
        _scat_one(ga, oa, src_hbm, dst_hbm, s, acc, scr)

    @pl.when(c == 1)
    def _():
        _scat_one(gb, ob, src_hbm, dst_hbm, s, acc, scr)


def _make_scat_kernel():
    return pl.kernel(
        _scat_body,
        out_type=(jax.ShapeDtypeStruct((N, HH), jnp.float32),
                  jax.ShapeDtypeStruct((N, HH), jnp.float32)),
        mesh=_mesh(),
        scratch_types=_scat_scratch(REM1),
    )


def _scat3_body(g, src_hbm, dst_hbm, z128, oa, ob, *scratch):
    c = lax.axis_index("c")
    s = lax.axis_index("s")
    acc = scratch[-1]
    scr = scratch[:-1]

    @pl.when(c == 0)
    def _():
        _tile_rows(s, lambda off, n: pltpu.sync_copy(
            g.at[pl.ds(off, n)], acc.at[pl.ds(off, n)]))

    @pl.when(c == 1)
    def _():
        _tile_rows(s, lambda off, n: pltpu.sync_copy(
            z128.at[pl.ds(off, n)], acc.at[pl.ds(off, n)]))

    plsc.subcore_barrier()
    _edge_pipe(g, acc, src_hbm, dst_hbm, (c * NS + s) * EPT2, FULL2, REM2, scr)
    plsc.subcore_barrier()

    @pl.when(c == 0)
    def _():
        _tile_rows(s, lambda off, n: pltpu.sync_copy(
            acc.at[pl.ds(off, n)], oa.at[pl.ds(off, n)]))

    @pl.when(c == 1)
    def _():
        _tile_rows(s, lambda off, n: pltpu.sync_copy(
            acc.at[pl.ds(off, n)], ob.at[pl.ds(off, n)]))


def _make_scat3_kernel():
    return pl.kernel(
        _scat3_body,
        out_type=(jax.ShapeDtypeStruct((N, HH), jnp.float32),
                  jax.ShapeDtypeStruct((N, HH), jnp.float32)),
        mesh=_mesh(),
        scratch_types=_scat_scratch(REM2),
    )


def _gather_body(lsm_hbm, idx_hbm, out, iv, rows, sem):
    c = lax.axis_index("c")
    s = lax.axis_index("s")
    w = c * NS + s
    for j in range(2):
        b = w * 160 + j * 80
        pltpu.sync_copy(idx_hbm.at[pl.ds(b, 80)], iv)
        pltpu.async_copy(lsm_hbm.at[iv], rows, sem).wait()
        pltpu.sync_copy(rows, out.at[pl.ds(b, 80)])


def _make_gather_kernel():
    return pl.kernel(
        _gather_body,
        out_type=jax.ShapeDtypeStruct((NIDXP, 128), jnp.float32),
        mesh=_mesh(),
        scratch_types=[
            pltpu.VMEM((80,), jnp.int32),
            pltpu.VMEM((80, 128), jnp.float32),
            pltpu.SemaphoreType.DMA,
        ],
    )


_R = 1000
_NR = N // _R


def _ds_of(hista, histb):
    deg = hista[:, 0:1] + histb[:, 0:1] + 1.0
    return lax.rsqrt(deg)


def _mm1_body(x_ref, w_ref, b_ref, ha_ref, hb_ref, oa_ref, ob_ref):
    ds = _ds_of(ha_ref[...], hb_ref[...])
    h = jnp.dot(x_ref[...], w_ref[...],
                preferred_element_type=jnp.float32) + b_ref[...]
    g = ds * h
    oa_ref[...] = g[:, :HH]
    ob_ref[...] = g[:, HH:]


def _mm2_body(oa_ref, ob_ref, w_ref, b_ref, ha_ref, hb_ref, ga_ref, gb_ref):
    ds = _ds_of(ha_ref[...], hb_ref[...])
    t = jax.nn.relu(ds * jnp.concatenate([oa_ref[...], ob_ref[...]], axis=1))
    h = jnp.dot(t, w_ref[...], preferred_element_type=jnp.float32) + b_ref[...]
    g = ds * h
    ga_ref[...] = g[:, :HH]
    gb_ref[...] = g[:, HH:]


def _mm3_body(oa_ref, ob_ref, w_ref, b_ref, ha_ref, hb_ref, out_ref):
    ds = _ds_of(ha_ref[...], hb_ref[...])
    t = jax.nn.relu(ds * jnp.concatenate([oa_ref[...], ob_ref[...]], axis=1))
    h = jnp.dot(t, w_ref[...], preferred_element_type=jnp.float32) + b_ref[...]
    out_ref[...] = jnp.concatenate(
        [ds * h, jnp.zeros((h.shape[0], 128 - CP), jnp.float32)], axis=1)


def _finish_body(oa_ref, ob_ref, ha_ref, hb_ref, out_ref):
    ds = _ds_of(ha_ref[...], hb_ref[...])
    zc = ds * (oa_ref[...][:, :C] + ob_ref[...][:, :C])
    m = jnp.max(zc, axis=1, keepdims=True)
    lse = jnp.log(jnp.sum(jnp.exp(zc - m), axis=1, keepdims=True)) + m
    out_ref[...] = jnp.concatenate(
        [zc - lse, jnp.zeros((zc.shape[0], 128 - C), jnp.float32)], axis=1)


def _hist_specs():
    return [
        pl.BlockSpec((_R, 16), lambda i: (i, 0)),
        pl.BlockSpec((_R, 16), lambda i: (_NR + i, 0)),
    ]


def _gpair_specs():
    return (pl.BlockSpec((_R, HH), lambda i: (i, 0)),
            jax.ShapeDtypeStruct((N, HH), jnp.float32))


def _mm1_call(x, w, b, hist):
    ospec, oshape = _gpair_specs()
    return pl.pallas_call(
        _mm1_body,
        grid=(_NR,),
        in_specs=[
            pl.BlockSpec((_R, D), lambda i: (i, 0)),
            pl.BlockSpec((D, H), lambda i: (0, 0)),
            pl.BlockSpec((1, H), lambda i: (0, 0)),
            *_hist_specs(),
        ],
        out_specs=(ospec, ospec),
        out_shape=(oshape, oshape),
    )(x, w, b.reshape(1, H), hist, hist)


def _mm2_call(oa, ob, w, b, hist):
    ospec, oshape = _gpair_specs()
    return pl.pallas_call(
        _mm2_body,
        grid=(_NR,),
        in_specs=[
            pl.BlockSpec((_R, HH), lambda i: (i, 0)),
            pl.BlockSpec((_R, HH), lambda i: (i, 0)),
            pl.BlockSpec((D, H), lambda i: (0, 0)),
            pl.BlockSpec((1, H), lambda i: (0, 0)),
            *_hist_specs(),
        ],
        out_specs=(ospec, ospec),
        out_shape=(oshape, oshape),
    )(oa, ob, w, b.reshape(1, H), hist, hist)


def _mm3_call(oa, ob, w, b, hist):
    return pl.pallas_call(
        _mm3_body,
        grid=(_NR,),
        in_specs=[
            pl.BlockSpec((_R, HH), lambda i: (i, 0)),
            pl.BlockSpec((_R, HH), lambda i: (i, 0)),
            pl.BlockSpec((D, CP), lambda i: (0, 0)),
            pl.BlockSpec((1, CP), lambda i: (0, 0)),
            *_hist_specs(),
        ],
        out_specs=pl.BlockSpec((_R, 128), lambda i: (i, 0)),
        out_shape=jax.ShapeDtypeStruct((N, 128), jnp.float32),
    )(oa, ob, w, b, hist, hist)


def _finish_call(oa, ob, hist):
    return pl.pallas_call(
        _finish_body,
        grid=(_NR,),
        in_specs=[
            pl.BlockSpec((_R, 128), lambda i: (i, 0)),
            pl.BlockSpec((_R, 128), lambda i: (i, 0)),
            *_hist_specs(),
        ],
        out_specs=pl.BlockSpec((_R, 128), lambda i: (i, 0)),
        out_shape=jax.ShapeDtypeStruct((N, 128), jnp.float32),
    )(oa, ob, hist, hist)


def kernel(x, edge_index, idx, W1, b1, W2, b2, Wd, bd):
    src = edge_index[0]
    dst = edge_index[1]
    zeros16 = jnp.zeros((N, 16), jnp.float32)
    zeros128 = jnp.zeros((N, 128), jnp.float32)
    Wdp = jnp.pad(Wd, ((0, 0), (0, CP - C)))
    bdp = jnp.pad(bd, (0, CP - C)).reshape(1, CP)
    idxp = jnp.pad(idx, (0, NIDXP - NIDX))

    hist = _make_deg_kernel()(dst, zeros16)
    scat = _make_scat_kernel()

    g1a, g1b = _mm1_call(x, W1, b1, hist)
    o1a, o1b = scat(g1a, g1b, src, dst)
    g2a, g2b = _mm2_call(o1a, o1b, W2, b2, hist)
    o2a, o2b = scat(g2a, g2b, src, dst)
    g3 = _mm3_call(o2a, o2b, Wdp, bdp, hist)
    o3a, o3b = _make_scat3_kernel()(g3, src, dst, zeros128)
    lsm = _finish_call(o3a, o3b, hist)
    out = _make_gather_kernel()(lsm, idxp)
    return out[:NIDX, :C]
